# Initial kernel scaffold; baseline (speedup 1.0000x reference)
import functools

import jax
import jax.numpy as jnp
from jax import lax
from jax.experimental import pallas as pl
from jax.experimental.pallas import tpu as pltpu
from jax.experimental.pallas import tpu_sc as plsc

N = 10000
D = 128
E = 320000
L = 3

NC = 2
NS = 16
NW = NC * NS
EPT = E // NW
CHUNK = 80
NCHUNKS = EPT // CHUNK
RPT = N // NS
ZROWS = 25

_mesh = plsc.VectorSubcoreMesh(core_axis_name="c", subcore_axis_name="s")


@functools.partial(
    pl.kernel,
    out_type=jax.ShapeDtypeStruct((NC, N, D), jnp.float32),
    mesh=_mesh,
    scratch_types=[
        pltpu.VMEM_SHARED((N, D), jnp.float32),
        pltpu.VMEM((CHUNK, D), jnp.float32),
        pltpu.VMEM((CHUNK,), jnp.int32),
        pltpu.VMEM((1, CHUNK), jnp.int32),
        pltpu.VMEM((ZROWS, D), jnp.float32),
        pltpu.SemaphoreType.DMA,
    ],
)
def _agg(x_hbm, src_hbm, dst_hbm, out_hbm, accum, rows_v, src_v, dst_v, zbuf, sem):
    c = lax.axis_index("c")
    s = lax.axis_index("s")
    wid = c * NS + s

    @pl.loop(0, ZROWS)
    def _zero_rows(r):
        @pl.loop(0, D // 16)
        def _zero_lanes(k):
            zbuf[r, pl.ds(k * 16, 16)] = jnp.zeros((16,), jnp.float32)

    @pl.loop(0, RPT // ZROWS)
    def _zero_stripe(t):
        pltpu.sync_copy(zbuf, accum.at[pl.ds(s * RPT + t * ZROWS, ZROWS)])

    plsc.subcore_barrier()

    base = wid * EPT

    @pl.loop(0, NCHUNKS)
    def _edges(j):
        off = base + j * CHUNK
        pltpu.sync_copy(src_hbm.at[pl.ds(off, CHUNK)], src_v)
        pltpu.sync_copy(dst_hbm.at[pl.ds(off, CHUNK)], dst_v.at[0])
        pltpu.async_copy(x_hbm.at[src_v], rows_v, sem).wait()
        pltpu.sync_copy(rows_v, accum.at[dst_v.at[0]], add=True)

    plsc.subcore_barrier()

    pltpu.sync_copy(accum.at[pl.ds(s * RPT, RPT)],
                    out_hbm.at[c, pl.ds(s * RPT, RPT)])


_TC_BLK = 2000


def _gin_tc_body(x_ref, p_ref, w_ref, b_ref, o_ref):
    h = x_ref[...] + p_ref[0] + p_ref[1]
    y = jnp.dot(h, w_ref[...], preferred_element_type=jnp.float32) + b_ref[...]
    o_ref[...] = jnp.maximum(y, 0.0)


def _gin_tc(x, p, w, b):
    return pl.pallas_call(
        _gin_tc_body,
        grid=(N // _TC_BLK,),
        in_specs=[
            pl.BlockSpec((_TC_BLK, D), lambda i: (i, 0)),
            pl.BlockSpec((NC, _TC_BLK, D), lambda i: (0, i, 0)),
            pl.BlockSpec((D, D), lambda i: (0, 0)),
            pl.BlockSpec((1, D), lambda i: (0, 0)),
        ],
        out_specs=pl.BlockSpec((_TC_BLK, D), lambda i: (i, 0)),
        out_shape=jax.ShapeDtypeStruct((N, D), jnp.float32),
    )(x, p, w, b)


def kernel(x, edge_indices, W0, b0, W1, b1, W2, b2):
    Ws = (W0, W1, W2)
    bs = (b0, b1, b2)
    for i in range(L):
        src = edge_indices[i, 1]
        dst = edge_indices[i, 0]
        p = _agg(x, src, dst)
        x = _gin_tc(x, p, Ws[i], bs[i].reshape(1, D))
    return x


# speedup vs baseline: 5.0314x; 5.0314x over previous
import functools

import jax
import jax.numpy as jnp
from jax import lax
from jax.experimental import pallas as pl
from jax.experimental.pallas import tpu as pltpu
from jax.experimental.pallas import tpu_sc as plsc

N = 10000
D = 128
E = 320000
L = 3

NC = 2
NS = 16
NW = NC * NS
EPT = E // NW
CHUNK = 80
NCHUNKS = EPT // CHUNK
NPAD = 10240
RPT = NPAD // NS
ZROWS = 32

_mesh = plsc.VectorSubcoreMesh(core_axis_name="c", subcore_axis_name="s")


@functools.partial(
    pl.kernel,
    out_type=jax.ShapeDtypeStruct((NC, NPAD, D), jnp.float32),
    mesh=_mesh,
    scratch_types=[
        pltpu.VMEM_SHARED((NPAD, D), jnp.float32),
        pltpu.VMEM((CHUNK, D), jnp.float32),
        pltpu.VMEM((CHUNK,), jnp.int32),
        pltpu.VMEM((1, CHUNK), jnp.int32),
        pltpu.VMEM((ZROWS, D), jnp.float32),
        pltpu.SemaphoreType.DMA,
    ],
)
def _agg(x_hbm, src_hbm, dst_hbm, out_hbm, accum, rows_v, src_v, dst_v, zbuf, sem):
    c = lax.axis_index("c")
    s = lax.axis_index("s")
    wid = c * NS + s

    @pl.loop(0, ZROWS)
    def _zero_rows(r):
        @pl.loop(0, D // 16)
        def _zero_lanes(k):
            zbuf[r, pl.ds(k * 16, 16)] = jnp.zeros((16,), jnp.float32)

    @pl.loop(0, RPT // ZROWS)
    def _zero_stripe(t):
        pltpu.sync_copy(zbuf, accum.at[pl.ds(s * RPT + t * ZROWS, ZROWS)])

    plsc.subcore_barrier()

    base = wid * EPT

    @pl.loop(0, NCHUNKS)
    def _edges(j):
        off = base + j * CHUNK
        pltpu.sync_copy(src_hbm.at[pl.ds(off, CHUNK)], src_v)
        pltpu.sync_copy(dst_hbm.at[pl.ds(off, CHUNK)], dst_v.at[0])
        pltpu.async_copy(x_hbm.at[src_v], rows_v, sem).wait()
        pltpu.sync_copy(rows_v, accum.at[dst_v.at[0]], add=True)

    plsc.subcore_barrier()

    pltpu.sync_copy(accum.at[pl.ds(s * RPT, RPT)],
                    out_hbm.at[c, pl.ds(s * RPT, RPT)])


_TC_BLK = 2000


def _gin_tc_body(x_ref, p_ref, w_ref, b_ref, o_ref):
    h = x_ref[...] + p_ref[0] + p_ref[1]
    y = jnp.dot(h, w_ref[...], preferred_element_type=jnp.float32) + b_ref[...]
    o_ref[...] = jnp.maximum(y, 0.0)


def _gin_tc(x, p, w, b):
    return pl.pallas_call(
        _gin_tc_body,
        grid=(N // _TC_BLK,),
        in_specs=[
            pl.BlockSpec((_TC_BLK, D), lambda i: (i, 0)),
            pl.BlockSpec((NC, _TC_BLK, D), lambda i: (0, i, 0)),
            pl.BlockSpec((D, D), lambda i: (0, 0)),
            pl.BlockSpec((1, D), lambda i: (0, 0)),
        ],
        out_specs=pl.BlockSpec((_TC_BLK, D), lambda i: (i, 0)),
        out_shape=jax.ShapeDtypeStruct((N, D), jnp.float32),
    )(x, p, w, b)


def kernel(x, edge_indices, W0, b0, W1, b1, W2, b2):
    Ws = (W0, W1, W2)
    bs = (b0, b1, b2)
    for i in range(L):
        src = edge_indices[i, 1]
        dst = edge_indices[i, 0]
        p = _agg(x, src, dst)
        x = _gin_tc(x, p, Ws[i], bs[i].reshape(1, D))
    return x
